# baseline (speedup 1.0000x reference)
import functools

import jax
import jax.numpy as jnp
from jax.experimental import pallas as pl
from jax.experimental.pallas import tpu as pltpu

_NBUF = 5


def _pick_chunk(n):
    for c in (2000, 200, 8):
        if n % c == 0 and (n // c) % _NBUF == 0:
            return c
    return n


def _pick_slice(chunk):
    for r in (80, 40, 8):
        if chunk % r == 0:
            return r
    return chunk


def _group_kernel(a_hbm, g_hbm, o_hbm, a_buf, g_buf, o_buf, ids, macc, iacc,
                  a_sem, g_sem, o_sem, *, n, m, chunk, ngroups):
    nchunks = n // chunk
    mcols = m // ngroups
    rows_sl = _pick_slice(chunk)
    nsub = rows_sl // 8
    qs_per_chunk = chunk // 8

    def a_copy(gg, c, s):
        return pltpu.make_async_copy(
            a_hbm.at[pl.ds(c * chunk, chunk), pl.ds(gg * mcols, mcols)],
            a_buf.at[s], a_sem.at[s])

    def g_copy(gg, c, s):
        return pltpu.make_async_copy(
            g_hbm.at[pl.ds(c * chunk, chunk), pl.ds(gg * mcols, mcols)],
            g_buf.at[s], g_sem.at[s])

    def o_copy(gg, c, s):
        return pltpu.make_async_copy(
            o_buf.at[s],
            o_hbm.at[pl.ds(c * chunk, chunk), pl.ds(gg * mcols, mcols)],
            o_sem.at[s])

    ids[...] = jax.lax.broadcasted_iota(jnp.int32, (chunk, mcols), 0)
    siota = jax.lax.broadcasted_iota(jnp.int32, (8, mcols), 0)

    def a_work(gg, i, pg):
        s = jax.lax.rem(i, _NBUF)
        a_copy(gg, i, s).wait()
        g_copy(gg, i, s).wait()

        def t_body(t, carry):
            m8, i8 = carry
            xs = (a_buf[s, pl.ds(t * rows_sl, rows_sl), :]
                  + g_buf[s, pl.ds(t * rows_sl, rows_sl), :])
            for u in range(nsub):
                row = xs[u * 8:(u + 1) * 8, :]
                q = i * qs_per_chunk + t * nsub + u
                isnew = row > m8
                i8 = jnp.where(isnew, q, i8)
                m8 = jnp.maximum(row, m8)
            return m8, i8

        m8, i8 = jax.lax.fori_loop(
            0, chunk // rows_sl, t_body,
            (jnp.full((8, mcols), -jnp.inf, jnp.float32),
             jnp.zeros((8, mcols), jnp.int32)))

        gi = i8 * 8 + siota
        bm = jnp.max(m8, axis=0, keepdims=True)
        bi = jnp.min(jnp.where(m8 >= bm, gi, jnp.int32(n)), axis=0,
                     keepdims=True)
        better = bm > macc[pg]
        iacc[pg] = jnp.where(better, bi, iacc[pg])
        macc[pg] = jnp.maximum(bm, macc[pg])

        @pl.when(i + _NBUF < nchunks)
        def _():
            a_copy(gg, i + _NBUF, s).start()
            g_copy(gg, i + _NBUF, s).start()

    def b_work(bg, i, pb):
        s = jax.lax.rem(i, _NBUF)
        if bg > 0:
            @pl.when(i < _NBUF)
            def _():
                o_copy(bg - 1, nchunks - _NBUF + i, s).wait()

        @pl.when(i >= _NBUF)
        def _():
            o_copy(bg, i - _NBUF, s).wait()

        rel = iacc[pb] - i * chunk
        o_buf[s] = (ids[...] == rel).astype(jnp.float32)
        o_copy(bg, i, s).start()

    for g in range(ngroups):
        pg = g % 2
        macc[pg] = jnp.full((1, mcols), -jnp.inf, jnp.float32)
        iacc[pg] = jnp.zeros((1, mcols), jnp.int32)
        for s in range(min(_NBUF, nchunks)):
            a_copy(g, s, s).start()
            g_copy(g, s, s).start()

        if g == 0:
            def body(i, carry):
                a_work(g, i, pg)
                return carry
        else:
            def body(i, carry, _g=g, _pg=pg):
                a_work(_g, i, _pg)
                b_work(_g - 1, i, 1 - _pg)
                return carry

        jax.lax.fori_loop(0, nchunks, body, 0)

    last = ngroups - 1

    def body_fin(i, carry):
        b_work(last, i, last % 2)
        return carry

    jax.lax.fori_loop(0, nchunks, body_fin, 0)
    for c in range(max(nchunks - _NBUF, 0), nchunks):
        o_copy(last, c, c % _NBUF).wait()


def kernel(alpha, gumbel, tau):
    del tau
    mm, kk = alpha.shape
    n, m = kk, mm
    chunk = _pick_chunk(n)
    ngroups = m // 256 if m % 256 == 0 else 1
    mcols = m // ngroups
    buf = lambda: pltpu.VMEM((_NBUF, chunk, mcols), jnp.float32)
    sem = lambda: pltpu.SemaphoreType.DMA((_NBUF,))
    out_t = pl.pallas_call(
        functools.partial(_group_kernel, n=n, m=m, chunk=chunk,
                          ngroups=ngroups),
        in_specs=[
            pl.BlockSpec(memory_space=pl.ANY),
            pl.BlockSpec(memory_space=pl.ANY),
        ],
        out_specs=pl.BlockSpec(memory_space=pl.ANY),
        out_shape=jax.ShapeDtypeStruct((n, m), jnp.float32),
        scratch_shapes=[
            buf(), buf(), buf(),
            pltpu.VMEM((chunk, mcols), jnp.int32),
            pltpu.VMEM((2, 1, mcols), jnp.float32),
            pltpu.VMEM((2, 1, mcols), jnp.int32),
            sem(), sem(), sem(),
        ],
    )(alpha.T, gumbel.T)
    return out_t.T
